# baseline (speedup 1.0000x reference)
import jax
import jax.numpy as jnp
from jax import lax
from jax.experimental import pallas as pl
from jax.experimental.pallas import tpu as pltpu
from jax.experimental.pallas import tpu_sc as plsc

_B = 64
_N = 100
_D = 256
_NC = 2
_NS = 16
_RPW = 256
_NW_USED = (_N * _B) // _RPW


def _sc_pe_body(emb_hbm, pe_hbm, idx_a, idx_b, rows_a, rows_b, sem_a, sem_b, sem_o):
    wid = lax.axis_index("s") * _NC + lax.axis_index("c")

    @pl.when(wid < _NW_USED)
    def _():
        base = wid * _RPW
        n0 = wid * (_RPW // _B)

        for c in range(8):
            idx_a[pl.ds(c * 16, 16)] = jnp.full((16,), n0 + c // 4, jnp.int32)
        for c in range(8, 16):
            idx_b[pl.ds((c - 8) * 16, 16)] = jnp.full((16,), n0 + c // 4, jnp.int32)
        ga = pltpu.async_copy(emb_hbm.at[idx_a], rows_a, sem_a)
        gb = pltpu.async_copy(emb_hbm.at[idx_b], rows_b, sem_b)
        ga.wait()
        oa = pltpu.async_copy(rows_a, pe_hbm.at[pl.ds(base, 128)], sem_o)
        gb.wait()
        ob = pltpu.async_copy(rows_b, pe_hbm.at[pl.ds(base + 128, 128)], sem_o)
        oa.wait()
        ob.wait()


def _tc_zeros_body(zt_ref):
    zt_ref[...] = jnp.zeros(zt_ref.shape, zt_ref.dtype)


def kernel(x, emb_table):
    mesh = plsc.VectorSubcoreMesh(core_axis_name="c", subcore_axis_name="s")
    sc_call = pl.kernel(
        _sc_pe_body,
        mesh=mesh,
        out_type=jax.ShapeDtypeStruct((_N * _B, _D), jnp.float32),
        scratch_types=[
            pltpu.VMEM((128,), jnp.int32),
            pltpu.VMEM((128,), jnp.int32),
            pltpu.VMEM((128, _D), jnp.float32),
            pltpu.VMEM((128, _D), jnp.float32),
            pltpu.SemaphoreType.DMA,
            pltpu.SemaphoreType.DMA,
            pltpu.SemaphoreType.DMA,
        ],
    )
    pet = sc_call(emb_table)

    nb = 20
    zt = pl.pallas_call(
        _tc_zeros_body,
        grid=(_N // nb,),
        out_specs=pl.BlockSpec((nb, _B, _D), lambda i: (i, 0, 0)),
        out_shape=jax.ShapeDtypeStruct((_N, _B, _D), jnp.float32),
    )()

    target = jnp.transpose(zt, (1, 0, 2))
    target_pe = jnp.transpose(pet.reshape(_N, _B, _D), (1, 0, 2))
    return (target, target_pe)


# speedup vs baseline: 3.8022x; 3.8022x over previous
import jax
import jax.numpy as jnp
from jax.experimental import pallas as pl

_B = 64
_N = 100
_D = 256


def _tc_body(emb_ref, zt_ref, pet_ref):
    zt_ref[...] = jnp.zeros(zt_ref.shape, zt_ref.dtype)
    pet_ref[...] = jnp.broadcast_to(emb_ref[...], pet_ref.shape)


def kernel(x, emb_table):
    nb = 10
    out_t = jax.ShapeDtypeStruct((_N, _B, _D), jnp.float32)
    zt, pet = pl.pallas_call(
        _tc_body,
        grid=(_N // nb,),
        in_specs=[pl.BlockSpec((nb, 1, _D), lambda i: (i, 0, 0))],
        out_specs=[
            pl.BlockSpec((nb, _B, _D), lambda i: (i, 0, 0)),
            pl.BlockSpec((nb, _B, _D), lambda i: (i, 0, 0)),
        ],
        out_shape=[out_t, out_t],
    )(emb_table.reshape(_N, 1, _D))
    return (jnp.transpose(zt, (1, 0, 2)), jnp.transpose(pet, (1, 0, 2)))


# speedup vs baseline: 4.9058x; 1.2903x over previous
import jax
import jax.numpy as jnp
from jax.experimental import pallas as pl

_B = 64
_N = 100
_D = 256


def _tc_body(emb_ref, zt_ref, pet_ref):
    zt_ref[...] = jnp.zeros(zt_ref.shape, zt_ref.dtype)
    pet_ref[...] = jnp.broadcast_to(emb_ref[...], pet_ref.shape)


def kernel(x, emb_table):
    nb = 50
    out_t = jax.ShapeDtypeStruct((_N, _B, _D), jnp.float32)
    zt, pet = pl.pallas_call(
        _tc_body,
        grid=(_N // nb,),
        in_specs=[pl.BlockSpec((nb, 1, _D), lambda i: (i, 0, 0))],
        out_specs=[
            pl.BlockSpec((nb, _B, _D), lambda i: (i, 0, 0)),
            pl.BlockSpec((nb, _B, _D), lambda i: (i, 0, 0)),
        ],
        out_shape=[out_t, out_t],
    )(emb_table.reshape(_N, 1, _D))
    return (jnp.transpose(zt, (1, 0, 2)), jnp.transpose(pet, (1, 0, 2)))
